# baseline (speedup 1.0000x reference)
import functools

import jax
import jax.numpy as jnp
from jax import lax
from jax.experimental import pallas as pl
from jax.experimental.pallas import tpu as pltpu
from jax.experimental.pallas import tpu_sc as plsc

ROWS = 128
COLS = 32768
LANES = 16
NUM_CORES = 2
NUM_SUBCORES = 16
NUM_WORKERS = NUM_CORES * NUM_SUBCORES
ROWS_PER_WORKER = ROWS // NUM_WORKERS
CHUNK = 4096
CHUNKS_PER_ROW = COLS // CHUNK
NACC = 8
P1_ELEMS = NACC * LANES
P1_STEPS = CHUNK // P1_ELEMS
NACC2 = 4
P2_ELEMS = NACC2 * LANES
P2_STEPS = CHUNK // P2_ELEMS
SLICES = 8
SLICE = COLS // SLICES

_INT_MAX = 2147483647


def _argmin_body(x_hbm, out_hbm, buf, out_v, sem0, sem1):
    sems = (sem0, sem1)
    wid = lax.axis_index("s") * NUM_CORES + lax.axis_index("c")
    row0 = wid * ROWS_PER_WORKER
    iota = lax.iota(jnp.int32, LANES)

    def start(r):
        row = row0 + r
        b = r & 1
        return [
            pltpu.async_copy(
                x_hbm.at[row, pl.ds(s * SLICE, SLICE)],
                buf.at[b, pl.ds(s * SLICE, SLICE)],
                sems[b])
            for s in range(SLICES)
        ]

    copies = [None, None]
    copies[0] = start(0)
    result_v = jnp.zeros((LANES,), jnp.int32)

    for r in range(ROWS_PER_WORKER):
        b = r & 1
        if r + 1 < ROWS_PER_WORKER:
            copies[(r + 1) & 1] = start(r + 1)
        for cp in copies[b]:
            cp.wait()

        chunkvecs = []
        for c in range(CHUNKS_PER_ROW):
            accs = tuple(jnp.full((LANES,), jnp.inf, jnp.float32)
                         for _ in range(NACC))

            def p1_body(i, carry, b=b, c=c):
                vs = list(carry)
                off = c * CHUNK + i * P1_ELEMS
                for u in range(NACC):
                    v = buf[b, pl.ds(off + u * LANES, LANES)]
                    vs[u] = jnp.minimum(vs[u], v)
                return tuple(vs)

            accs = plsc.parallel_loop(0, P1_STEPS, 1, carry=accs)(p1_body)
            accs = list(accs)
            while len(accs) > 1:
                accs = [jnp.minimum(accs[j], accs[j + 1])
                        for j in range(0, len(accs), 2)]
            chunkvecs.append(accs[0])

        mv = chunkvecs[0]
        for c in range(1, CHUNKS_PER_ROW):
            mv = jnp.minimum(mv, chunkvecs[c])
        rowmin = jnp.min(mv)

        chunk_id = jnp.int32(0)
        for c in reversed(range(CHUNKS_PER_ROW)):
            has = jnp.any(chunkvecs[c] == rowmin)
            chunk_id = jnp.where(has, jnp.int32(c), chunk_id)

        base = chunk_id * CHUNK
        minis = tuple(jnp.full((LANES,), _INT_MAX, jnp.int32)
                      for _ in range(NACC2))

        def p2_body(i, carry, b=b):
            ms = list(carry)
            off = base + i * P2_ELEMS
            for u in range(NACC2):
                v = buf[b, pl.ds(off + u * LANES, LANES)]
                idxv = iota + (off + u * LANES)
                cand = jnp.where(v == rowmin, idxv, jnp.int32(_INT_MAX))
                ms[u] = jnp.minimum(ms[u], cand)
            return tuple(ms)

        minis = plsc.parallel_loop(0, P2_STEPS, 1, carry=minis)(p2_body)
        minis = list(minis)
        while len(minis) > 1:
            minis = [jnp.minimum(minis[j], minis[j + 1])
                     for j in range(0, len(minis), 2)]
        rowarg = jnp.min(minis[0])
        result_v = jnp.where(iota == r, rowarg, result_v)

    out_v[...] = result_v
    pltpu.sync_copy(out_v, out_hbm.at[wid])


def kernel(x):
    mesh = plsc.VectorSubcoreMesh(core_axis_name="c", subcore_axis_name="s")
    k = functools.partial(
        pl.kernel,
        mesh=mesh,
        out_type=jax.ShapeDtypeStruct((NUM_WORKERS, LANES), jnp.int32),
        scratch_types=[
            pltpu.VMEM((2, COLS), jnp.float32),
            pltpu.VMEM((LANES,), jnp.int32),
            pltpu.SemaphoreType.DMA,
            pltpu.SemaphoreType.DMA,
        ],
        compiler_params=pltpu.CompilerParams(needs_layout_passes=False),
    )(_argmin_body)
    y = k(x)
    return y[:, :ROWS_PER_WORKER].reshape(ROWS, 1)


# speedup vs baseline: 1.1725x; 1.1725x over previous
import functools

import jax
import jax.numpy as jnp
from jax import lax
from jax.experimental import pallas as pl
from jax.experimental.pallas import tpu as pltpu
from jax.experimental.pallas import tpu_sc as plsc

ROWS = 128
COLS = 32768
LANES = 16
NUM_CORES = 2
NUM_SUBCORES = 16
NUM_WORKERS = NUM_CORES * NUM_SUBCORES
TROW = 8
NUM_TROWS = ROWS // TROW
HALF = COLS // 2
CHUNK = 4096
CHUNKS = HALF // CHUNK
STEPS = CHUNK // LANES

_INT_MAX = 2147483647


def _argmin_body(x_hbm, val_hbm, idx_hbm, buf, outv_val, outv_idx,
                 sem0, sem1):
    sems = (sem0, sem1)
    wid = lax.axis_index("s") * NUM_CORES + lax.axis_index("c")
    trow = wid // 2
    half = wid % 2
    row0 = trow * TROW
    col0 = half * HALF
    iota = lax.iota(jnp.int32, LANES)

    def start(c):
        return pltpu.async_copy(
            x_hbm.at[pl.ds(row0, TROW), pl.ds(col0 + c * CHUNK, CHUNK)],
            buf.at[c & 1], sems[c & 1])

    copies = [None, None]
    copies[0] = start(0)

    accv = [jnp.full((LANES,), jnp.inf, jnp.float32) for _ in range(TROW)]
    accs = [jnp.zeros((LANES,), jnp.int32) for _ in range(TROW)]

    for c in range(CHUNKS):
        b = c & 1
        if c + 1 < CHUNKS:
            copies[(c + 1) & 1] = start(c + 1)
        copies[b].wait()

        def p1_body(k, carry, b=b, c=c):
            vs = list(carry[0])
            ss = list(carry[1])
            stamp = jnp.zeros((LANES,), jnp.int32) + (c * STEPS + k)
            for s in range(TROW):
                v = buf[b, s, pl.ds(k * LANES, LANES)]
                m = v < vs[s]
                vs[s] = jnp.where(m, v, vs[s])
                ss[s] = jnp.where(m, stamp, ss[s])
            return (tuple(vs), tuple(ss))

        accv_t, accs_t = plsc.parallel_loop(
            0, STEPS, 1, carry=(tuple(accv), tuple(accs)))(p1_body)
        accv = list(accv_t)
        accs = list(accs_t)

    val_v = jnp.zeros((LANES,), jnp.float32)
    idx_v = jnp.zeros((LANES,), jnp.int32)
    for s in range(TROW):
        rowmin = jnp.min(accv[s])
        colvec = accs[s] * LANES + iota + col0
        cand = jnp.where(accv[s] == rowmin, colvec, jnp.int32(_INT_MAX))
        rowidx = jnp.min(cand)
        val_v = jnp.where(iota == s, rowmin, val_v)
        idx_v = jnp.where(iota == s, rowidx, idx_v)

    outv_val[...] = val_v
    outv_idx[...] = idx_v
    pltpu.sync_copy(outv_val, val_hbm.at[wid])
    pltpu.sync_copy(outv_idx, idx_hbm.at[wid])


def kernel(x):
    mesh = plsc.VectorSubcoreMesh(core_axis_name="c", subcore_axis_name="s")
    k = functools.partial(
        pl.kernel,
        mesh=mesh,
        out_type=(
            jax.ShapeDtypeStruct((NUM_WORKERS, LANES), jnp.float32),
            jax.ShapeDtypeStruct((NUM_WORKERS, LANES), jnp.int32),
        ),
        scratch_types=[
            pltpu.VMEM((2, TROW, CHUNK), jnp.float32),
            pltpu.VMEM((LANES,), jnp.float32),
            pltpu.VMEM((LANES,), jnp.int32),
            pltpu.SemaphoreType.DMA,
            pltpu.SemaphoreType.DMA,
        ],
        compiler_params=pltpu.CompilerParams(needs_layout_passes=False),
    )(_argmin_body)
    vals, idxs = k(x)
    v = vals[:, :TROW].reshape(NUM_TROWS, 2, TROW)
    i = idxs[:, :TROW].reshape(NUM_TROWS, 2, TROW)
    take_hi = v[:, 1] < v[:, 0]
    y = jnp.where(take_hi, i[:, 1], i[:, 0])
    return y.reshape(ROWS, 1)


# speedup vs baseline: 1.1913x; 1.0160x over previous
import functools

import jax
import jax.numpy as jnp
from jax import lax
from jax.experimental import pallas as pl
from jax.experimental.pallas import tpu as pltpu
from jax.experimental.pallas import tpu_sc as plsc

ROWS = 128
COLS = 32768
LANES = 16
NUM_CORES = 2
NUM_SUBCORES = 16
NUM_WORKERS = NUM_CORES * NUM_SUBCORES
TROW = 8
NUM_TROWS = ROWS // TROW
HALF = COLS // 2
CHUNK = 2048
CHUNKS = HALF // CHUNK
STEPS = CHUNK // LANES
NBUF = 4

_INT_MAX = 2147483647


def _argmin_body(x_hbm, val_hbm, idx_hbm, buf, outv_val, outv_idx,
                 sem0, sem1, sem2, sem3):
    sems = (sem0, sem1, sem2, sem3)
    wid = lax.axis_index("s") * NUM_CORES + lax.axis_index("c")
    trow = wid // 2
    half = wid % 2
    row0 = trow * TROW
    col0 = half * HALF
    iota = lax.iota(jnp.int32, LANES)

    def start(c):
        return pltpu.async_copy(
            x_hbm.at[pl.ds(row0, TROW), pl.ds(col0 + c * CHUNK, CHUNK)],
            buf.at[c % NBUF], sems[c % NBUF])

    copies = [None] * NBUF
    for c in range(NBUF - 1):
        copies[c] = start(c)

    accv = [jnp.full((LANES,), jnp.inf, jnp.float32) for _ in range(TROW)]
    accs = [jnp.zeros((LANES,), jnp.int32) for _ in range(TROW)]

    for c in range(CHUNKS):
        b = c % NBUF
        if c + NBUF - 1 < CHUNKS:
            copies[(c + NBUF - 1) % NBUF] = start(c + NBUF - 1)
        copies[b].wait()

        def p1_body(k, carry, b=b, c=c):
            vs = list(carry[0])
            ss = list(carry[1])
            stamp = jnp.zeros((LANES,), jnp.int32) + (c * STEPS + k)
            for s in range(TROW):
                v = buf[b, s, pl.ds(k * LANES, LANES)]
                m = v < vs[s]
                vs[s] = jnp.where(m, v, vs[s])
                ss[s] = jnp.where(m, stamp, ss[s])
            return (tuple(vs), tuple(ss))

        accv_t, accs_t = plsc.parallel_loop(
            0, STEPS, 1, carry=(tuple(accv), tuple(accs)))(p1_body)
        accv = list(accv_t)
        accs = list(accs_t)

    val_v = jnp.zeros((LANES,), jnp.float32)
    idx_v = jnp.zeros((LANES,), jnp.int32)
    for s in range(TROW):
        rowmin = jnp.min(accv[s])
        colvec = accs[s] * LANES + iota + col0
        cand = jnp.where(accv[s] == rowmin, colvec, jnp.int32(_INT_MAX))
        rowidx = jnp.min(cand)
        val_v = jnp.where(iota == s, rowmin, val_v)
        idx_v = jnp.where(iota == s, rowidx, idx_v)

    outv_val[...] = val_v
    outv_idx[...] = idx_v
    pltpu.sync_copy(outv_val, val_hbm.at[wid])
    pltpu.sync_copy(outv_idx, idx_hbm.at[wid])


def kernel(x):
    mesh = plsc.VectorSubcoreMesh(core_axis_name="c", subcore_axis_name="s")
    k = functools.partial(
        pl.kernel,
        mesh=mesh,
        out_type=(
            jax.ShapeDtypeStruct((NUM_WORKERS, LANES), jnp.float32),
            jax.ShapeDtypeStruct((NUM_WORKERS, LANES), jnp.int32),
        ),
        scratch_types=[
            pltpu.VMEM((NBUF, TROW, CHUNK), jnp.float32),
            pltpu.VMEM((LANES,), jnp.float32),
            pltpu.VMEM((LANES,), jnp.int32),
            pltpu.SemaphoreType.DMA,
            pltpu.SemaphoreType.DMA,
            pltpu.SemaphoreType.DMA,
            pltpu.SemaphoreType.DMA,
        ],
        compiler_params=pltpu.CompilerParams(needs_layout_passes=False),
    )(_argmin_body)
    vals, idxs = k(x)
    v = vals[:, :TROW].reshape(NUM_TROWS, 2, TROW)
    i = idxs[:, :TROW].reshape(NUM_TROWS, 2, TROW)
    take_hi = v[:, 1] < v[:, 0]
    y = jnp.where(take_hi, i[:, 1], i[:, 0])
    return y.reshape(ROWS, 1)
